# baseline (speedup 1.0000x reference)
import jax
import jax.numpy as jnp
from jax import lax
from jax.experimental import pallas as pl
from jax.experimental.pallas import tpu as pltpu
from jax.experimental.pallas import tpu_sc as plsc

_B = 2048
_IN = 4096
_OUT = 8192
_L = 16

_NC = 2
_NS = 16
_NW = _NC * _NS
_ROWS_W = _B // _NW
_RB = 8
_NRB = _ROWS_W // _RB
_C = 1024
_NCH = _OUT // _C
_G = _C // _L

_MT = [
    [0, 0, 0, 0, 0, 0, 0, 0, 1, 1, 1, 1, 1, 1, 1, 1],
    [0, 0, 1, 1, 0, 0, 1, 1, -1, -1, 0, 0, -1, -1, 0, 0],
    [0, 0, 0, 0, 1, 1, 1, 1, -1, -1, -1, -1, 0, 0, 0, 0],
    [0, 1, -1, 0, -1, 0, -2, -1, 1, 2, 0, 1, 0, 1, -1, 0],
]


def _coeff_body(w_ref, mt_ref, ia_ref, ib_ref, out_ref, iab_ref):
    w = w_ref[...]
    m = jnp.max(w, axis=-1, keepdims=True)
    e = jnp.exp(w - m)
    p = e / jnp.sum(e, axis=-1, keepdims=True)
    out_ref[...] = lax.dot_general(
        mt_ref[...], p, (((1,), (1,)), ((), ())),
        preferred_element_type=jnp.float32,
        precision=lax.Precision.HIGHEST,
    )
    iab_ref[...] = ia_ref[...] + (ib_ref[...] << 12)


def _sc_body(x_hbm, iab_hbm, co_hbm,
             out_hbm, xbuf, iabbuf, c0buf, cabuf, cbbuf, cabbuf, obuf,
             xsem, osems):
    wid = lax.axis_index("s") * _NC + lax.axis_index("c")
    r0 = wid * _ROWS_W

    for r in range(_RB):
        pltpu.async_copy(
            x_hbm.at[r0 + r], xbuf.at[pl.ds(r * _IN, _IN)], xsem)
    pltpu.sync_copy(iab_hbm.at[0], iabbuf)
    pltpu.sync_copy(co_hbm.at[0], c0buf)
    pltpu.sync_copy(co_hbm.at[1], cabuf)
    pltpu.sync_copy(co_hbm.at[2], cbbuf)
    pltpu.sync_copy(co_hbm.at[3], cabbuf)

    def rb_body(rb, carry0):
        xslot = lax.rem(rb, 2)
        row = r0 + rb * _RB
        xofs = xslot * (_RB * _IN)
        for r in range(_RB):
            pltpu.make_async_copy(
                x_hbm.at[row + r],
                xbuf.at[pl.ds(xofs + r * _IN, _IN)], xsem).wait()

        @pl.when(rb < _NRB - 1)
        def _():
            nofs = (1 - xslot) * (_RB * _IN)
            for r in range(_RB):
                pltpu.async_copy(
                    x_hbm.at[row + _RB + r],
                    xbuf.at[pl.ds(nofs + r * _IN, _IN)], xsem)

        def jc_body(jc, carry1):
            oslot = lax.rem(jc, 2)
            j0 = jc * _C

            @pl.when(rb * _NCH + jc >= 2)
            def _():
                pltpu.make_async_copy(
                    obuf.at[oslot],
                    out_hbm.at[pl.ds(row, _RB), pl.ds(j0, _C)],
                    osems.at[oslot]).wait()

            @plsc.parallel_loop(0, _G, unroll=8)
            def g_body(g):
                gofs = j0 + g * _L
                iab = iabbuf[pl.ds(gofs, _L)]
                ia = jnp.bitwise_and(iab, 4095)
                ib = jnp.right_shift(iab, 12)
                c0 = c0buf[pl.ds(gofs, _L)]
                ca = cabuf[pl.ds(gofs, _L)]
                cb = cbbuf[pl.ds(gofs, _L)]
                cab = cabbuf[pl.ds(gofs, _L)]
                for r in range(_RB):
                    xrow = xbuf.at[pl.ds(xofs + r * _IN, _IN)]
                    av = plsc.load_gather(xrow, [ia])
                    bv = plsc.load_gather(xrow, [ib])
                    o = c0 + ca * av + cb * bv + cab * (av * bv)
                    obuf[oslot, r, pl.ds(g * _L, _L)] = o

            pltpu.async_copy(
                obuf.at[oslot],
                out_hbm.at[pl.ds(row, _RB), pl.ds(j0, _C)],
                osems.at[oslot])
            return carry1

        lax.fori_loop(0, _NCH, jc_body, 0)
        return carry0

    lax.fori_loop(0, _NRB, rb_body, 0)

    for oslot in range(2):
        pltpu.make_async_copy(
            obuf.at[oslot],
            out_hbm.at[pl.ds(r0, _RB), pl.ds(0, _C)],
            osems.at[oslot]).wait()


def kernel(x, weights, idx_a, idx_b):
    coeffs, iab = pl.pallas_call(
        _coeff_body,
        out_shape=[
            jax.ShapeDtypeStruct((4, _OUT), jnp.float32),
            jax.ShapeDtypeStruct((1, _OUT), jnp.int32),
        ],
    )(weights, jnp.asarray(_MT, dtype=jnp.float32),
      idx_a.reshape(1, _OUT), idx_b.reshape(1, _OUT))
    sc = pl.kernel(
        _sc_body,
        out_type=jax.ShapeDtypeStruct((_B, _OUT), jnp.float32),
        mesh=plsc.VectorSubcoreMesh(core_axis_name="c", subcore_axis_name="s"),
        compiler_params=pltpu.CompilerParams(needs_layout_passes=False),
        scratch_types=[
            pltpu.VMEM((2 * _RB * _IN,), jnp.float32),
            pltpu.VMEM((_OUT,), jnp.int32),
            pltpu.VMEM((_OUT,), jnp.float32),
            pltpu.VMEM((_OUT,), jnp.float32),
            pltpu.VMEM((_OUT,), jnp.float32),
            pltpu.VMEM((_OUT,), jnp.float32),
            pltpu.VMEM((2, _RB, _C), jnp.float32),
            pltpu.SemaphoreType.DMA,
            pltpu.SemaphoreType.DMA((2,)),
        ],
    )
    return sc(x, iab, coeffs)


# speedup vs baseline: 2.2955x; 2.2955x over previous
import jax
import jax.numpy as jnp
from jax import lax
from jax.experimental import pallas as pl
from jax.experimental.pallas import tpu as pltpu
from jax.experimental.pallas import tpu_sc as plsc

_B = 2048
_IN = 4096
_OUT = 8192
_L = 16

_NC = 2
_NS = 16
_NW = _NC * _NS
_ROWS_W = _B // _NW
_RB = 8
_NRB = _ROWS_W // _RB
_C = 1024
_NCH = _OUT // _C
_G = _C // _L

_MT = [
    [0, 0, 0, 0, 0, 0, 0, 0, 1, 1, 1, 1, 1, 1, 1, 1],
    [0, 0, 1, 1, 0, 0, 1, 1, -1, -1, 0, 0, -1, -1, 0, 0],
    [0, 0, 0, 0, 1, 1, 1, 1, -1, -1, -1, -1, 0, 0, 0, 0],
    [0, 1, -1, 0, -1, 0, -2, -1, 1, 2, 0, 1, 0, 1, -1, 0],
]


def _coeff_body(w_ref, mt_ref, ia_ref, ib_ref, out_ref, iab_ref):
    w = w_ref[...]
    m = jnp.max(w, axis=-1, keepdims=True)
    e = jnp.exp(w - m)
    p = e / jnp.sum(e, axis=-1, keepdims=True)
    out_ref[...] = lax.dot_general(
        mt_ref[...], p, (((1,), (1,)), ((), ())),
        preferred_element_type=jnp.float32,
        precision=lax.Precision.HIGHEST,
    )
    iab_ref[...] = ia_ref[...] + (ib_ref[...] << 12)


def _sc_body(x_hbm, iab_hbm, co_hbm,
             out_hbm, xbuf, iabbuf, c0buf, cabuf, cbbuf, cabbuf, obuf,
             xsem, osems):
    wid = lax.axis_index("s") * _NC + lax.axis_index("c")
    r0 = wid * _ROWS_W

    for r in range(_RB):
        pltpu.async_copy(
            x_hbm.at[r0 + r], xbuf.at[pl.ds(r * _IN, _IN)], xsem)
    pltpu.sync_copy(iab_hbm.at[0], iabbuf)
    pltpu.sync_copy(co_hbm.at[0], c0buf)
    pltpu.sync_copy(co_hbm.at[1], cabuf)
    pltpu.sync_copy(co_hbm.at[2], cbbuf)
    pltpu.sync_copy(co_hbm.at[3], cabbuf)

    def rb_body(rb, carry0):
        xslot = lax.rem(rb, 2)
        row = r0 + rb * _RB
        xofs = xslot * (_RB * _IN)
        for r in range(_RB):
            pltpu.make_async_copy(
                x_hbm.at[row + r],
                xbuf.at[pl.ds(xofs + r * _IN, _IN)], xsem).wait()

        @pl.when(rb < _NRB - 1)
        def _():
            nofs = (1 - xslot) * (_RB * _IN)
            for r in range(_RB):
                pltpu.async_copy(
                    x_hbm.at[row + _RB + r],
                    xbuf.at[pl.ds(nofs + r * _IN, _IN)], xsem)

        def jc_body(jc, carry1):
            oslot = lax.rem(jc, 2)
            j0 = jc * _C

            @pl.when(rb * _NCH + jc >= 2)
            def _():
                pltpu.make_async_copy(
                    obuf.at[oslot],
                    out_hbm.at[pl.ds(row, _RB), pl.ds(j0, _C)],
                    osems.at[oslot]).wait()

            @plsc.parallel_loop(0, _G, unroll=2)
            def g_body(g):
                gofs = j0 + g * _L
                iab = iabbuf[pl.ds(gofs, _L)]
                ia = jnp.bitwise_and(iab, 4095)
                ib = jnp.right_shift(iab, 12)
                c0 = c0buf[pl.ds(gofs, _L)]
                ca = cabuf[pl.ds(gofs, _L)]
                cb = cbbuf[pl.ds(gofs, _L)]
                cab = cabbuf[pl.ds(gofs, _L)]
                for r in range(_RB):
                    xrow = xbuf.at[pl.ds(xofs + r * _IN, _IN)]
                    av = plsc.load_gather(xrow, [ia])
                    bv = plsc.load_gather(xrow, [ib])
                    o = c0 + ca * av + cb * bv + cab * (av * bv)
                    obuf[oslot, r, pl.ds(g * _L, _L)] = o

            pltpu.async_copy(
                obuf.at[oslot],
                out_hbm.at[pl.ds(row, _RB), pl.ds(j0, _C)],
                osems.at[oslot])
            return carry1

        lax.fori_loop(0, _NCH, jc_body, 0)
        return carry0

    lax.fori_loop(0, _NRB, rb_body, 0)

    for oslot in range(2):
        pltpu.make_async_copy(
            obuf.at[oslot],
            out_hbm.at[pl.ds(r0, _RB), pl.ds(0, _C)],
            osems.at[oslot]).wait()


def kernel(x, weights, idx_a, idx_b):
    coeffs, iab = pl.pallas_call(
        _coeff_body,
        out_shape=[
            jax.ShapeDtypeStruct((4, _OUT), jnp.float32),
            jax.ShapeDtypeStruct((1, _OUT), jnp.int32),
        ],
    )(weights, jnp.asarray(_MT, dtype=jnp.float32),
      idx_a.reshape(1, _OUT), idx_b.reshape(1, _OUT))
    sc = pl.kernel(
        _sc_body,
        out_type=jax.ShapeDtypeStruct((_B, _OUT), jnp.float32),
        mesh=plsc.VectorSubcoreMesh(core_axis_name="c", subcore_axis_name="s"),
        compiler_params=pltpu.CompilerParams(needs_layout_passes=False),
        scratch_types=[
            pltpu.VMEM((2 * _RB * _IN,), jnp.float32),
            pltpu.VMEM((_OUT,), jnp.int32),
            pltpu.VMEM((_OUT,), jnp.float32),
            pltpu.VMEM((_OUT,), jnp.float32),
            pltpu.VMEM((_OUT,), jnp.float32),
            pltpu.VMEM((_OUT,), jnp.float32),
            pltpu.VMEM((2, _RB, _C), jnp.float32),
            pltpu.SemaphoreType.DMA,
            pltpu.SemaphoreType.DMA((2,)),
        ],
    )
    return sc(x, iab, coeffs)


# speedup vs baseline: 2.4752x; 1.0783x over previous
import jax
import jax.numpy as jnp
from jax import lax
from jax.experimental import pallas as pl
from jax.experimental.pallas import tpu as pltpu
from jax.experimental.pallas import tpu_sc as plsc

_B = 2048
_IN = 4096
_OUT = 8192
_L = 16

_NC = 2
_NS = 16
_NW = _NC * _NS
_ROWS_W = _B // _NW
_RB = 8
_NRB = _ROWS_W // _RB
_C = 1024
_NCH = _OUT // _C
_G = _C // _L

_MT = [
    [0, 0, 0, 0, 0, 0, 0, 0, 1, 1, 1, 1, 1, 1, 1, 1],
    [0, 0, 1, 1, 0, 0, 1, 1, -1, -1, 0, 0, -1, -1, 0, 0],
    [0, 0, 0, 0, 1, 1, 1, 1, -1, -1, -1, -1, 0, 0, 0, 0],
    [0, 1, -1, 0, -1, 0, -2, -1, 1, 2, 0, 1, 0, 1, -1, 0],
]


def _coeff_body(w_ref, mt_ref, ia_ref, ib_ref, out_ref, iab_ref):
    w = w_ref[...]
    m = jnp.max(w, axis=-1, keepdims=True)
    e = jnp.exp(w - m)
    p = e / jnp.sum(e, axis=-1, keepdims=True)
    out_ref[...] = lax.dot_general(
        mt_ref[...], p, (((1,), (1,)), ((), ())),
        preferred_element_type=jnp.float32,
        precision=lax.Precision.HIGHEST,
    )
    iab_ref[...] = ia_ref[...] + (ib_ref[...] << 12)


def _sc_body(x_hbm, iab_hbm, co_hbm,
             out_hbm, xbuf, iabbuf, c0buf, cabuf, cbbuf, cabbuf, obuf,
             xsem, osems):
    wid = lax.axis_index("s") * _NC + lax.axis_index("c")
    r0 = wid * _ROWS_W

    for r in range(_RB):
        pltpu.async_copy(
            x_hbm.at[r0 + r], xbuf.at[pl.ds(r * _IN, _IN)], xsem)
    pltpu.sync_copy(iab_hbm.at[0], iabbuf)
    pltpu.sync_copy(co_hbm.at[0], c0buf)
    pltpu.sync_copy(co_hbm.at[1], cabuf)
    pltpu.sync_copy(co_hbm.at[2], cbbuf)
    pltpu.sync_copy(co_hbm.at[3], cabbuf)

    def rb_body(rb, carry0):
        xslot = lax.rem(rb, 2)
        row = r0 + rb * _RB
        xofs = xslot * (_RB * _IN)
        for r in range(_RB):
            pltpu.make_async_copy(
                x_hbm.at[row + r],
                xbuf.at[pl.ds(xofs + r * _IN, _IN)], xsem).wait()

        @pl.when(rb < _NRB - 1)
        def _():
            nofs = (1 - xslot) * (_RB * _IN)
            for r in range(_RB):
                pltpu.async_copy(
                    x_hbm.at[row + _RB + r],
                    xbuf.at[pl.ds(nofs + r * _IN, _IN)], xsem)

        def jc_body(jc, carry1):
            oslot = lax.rem(jc, 2)
            j0 = jc * _C

            @pl.when(rb * _NCH + jc >= 2)
            def _():
                pltpu.make_async_copy(
                    obuf.at[oslot],
                    out_hbm.at[pl.ds(row, _RB), pl.ds(j0, _C)],
                    osems.at[oslot]).wait()

            @plsc.parallel_loop(0, _G, unroll=1)
            def g_body(g):
                gofs = j0 + g * _L
                iab = iabbuf[pl.ds(gofs, _L)]
                ia = jnp.bitwise_and(iab, 4095)
                ib = jnp.right_shift(iab, 12)
                c0 = c0buf[pl.ds(gofs, _L)]
                ca = cabuf[pl.ds(gofs, _L)]
                cb = cbbuf[pl.ds(gofs, _L)]
                cab = cabbuf[pl.ds(gofs, _L)]
                for r in range(_RB):
                    xrow = xbuf.at[pl.ds(xofs + r * _IN, _IN)]
                    av = plsc.load_gather(xrow, [ia])
                    bv = plsc.load_gather(xrow, [ib])
                    o = c0 + ca * av + cb * bv + cab * (av * bv)
                    obuf[oslot, r, pl.ds(g * _L, _L)] = o

            pltpu.async_copy(
                obuf.at[oslot],
                out_hbm.at[pl.ds(row, _RB), pl.ds(j0, _C)],
                osems.at[oslot])
            return carry1

        lax.fori_loop(0, _NCH, jc_body, 0)
        return carry0

    lax.fori_loop(0, _NRB, rb_body, 0)

    for oslot in range(2):
        pltpu.make_async_copy(
            obuf.at[oslot],
            out_hbm.at[pl.ds(r0, _RB), pl.ds(0, _C)],
            osems.at[oslot]).wait()


def kernel(x, weights, idx_a, idx_b):
    coeffs, iab = pl.pallas_call(
        _coeff_body,
        out_shape=[
            jax.ShapeDtypeStruct((4, _OUT), jnp.float32),
            jax.ShapeDtypeStruct((1, _OUT), jnp.int32),
        ],
    )(weights, jnp.asarray(_MT, dtype=jnp.float32),
      idx_a.reshape(1, _OUT), idx_b.reshape(1, _OUT))
    sc = pl.kernel(
        _sc_body,
        out_type=jax.ShapeDtypeStruct((_B, _OUT), jnp.float32),
        mesh=plsc.VectorSubcoreMesh(core_axis_name="c", subcore_axis_name="s"),
        compiler_params=pltpu.CompilerParams(needs_layout_passes=False),
        scratch_types=[
            pltpu.VMEM((2 * _RB * _IN,), jnp.float32),
            pltpu.VMEM((_OUT,), jnp.int32),
            pltpu.VMEM((_OUT,), jnp.float32),
            pltpu.VMEM((_OUT,), jnp.float32),
            pltpu.VMEM((_OUT,), jnp.float32),
            pltpu.VMEM((_OUT,), jnp.float32),
            pltpu.VMEM((2, _RB, _C), jnp.float32),
            pltpu.SemaphoreType.DMA,
            pltpu.SemaphoreType.DMA((2,)),
        ],
    )
    return sc(x, iab, coeffs)
